# baseline (speedup 1.0000x reference)
import functools

import jax
import jax.numpy as jnp
from jax import lax
from jax.experimental import pallas as pl
from jax.experimental.pallas import tpu as pltpu
from jax.experimental.pallas import tpu_sc as plsc

DIM = 2048
HALF = 1024
S_TILE = 512


def _gather_body(yang_hbm, yin_hbm, cyc_hbm, yang_v, yin_v, cyc_v):
    c = lax.axis_index("c")
    s = lax.axis_index("s")
    wid = s * 2 + c

    @pl.when(wid == 0)
    def _():
        pltpu.sync_copy(yang_hbm, yang_v)
        pltpu.sync_copy(yin_hbm, yin_v)
        for i in range(12):
            i6 = (i + 6) % 12

            @plsc.parallel_loop(0, HALF // 16, unroll=8)
            def _(k, i=i, i6=i6):
                off = k * 16
                cyc_v[i, pl.ds(off, 16)] = yang_v[i, pl.ds(off, 16)]
                cyc_v[i, pl.ds(HALF + off, 16)] = yin_v[i6, pl.ds(off, 16)]

        pltpu.sync_copy(cyc_v, cyc_hbm)


def _sc_gather_cyc(yang_wheel, yin_wheel):
    mesh = plsc.VectorSubcoreMesh(core_axis_name="c", subcore_axis_name="s",
                                  num_cores=1, num_subcores=1)
    k = functools.partial(
        pl.kernel,
        mesh=mesh,
        out_type=jax.ShapeDtypeStruct((12, DIM), jnp.float32),
        scratch_types=[
            pltpu.VMEM((12, HALF), jnp.float32),
            pltpu.VMEM((12, HALF), jnp.float32),
            pltpu.VMEM((12, DIM), jnp.float32),
        ],
    )(_gather_body)
    return k(yang_wheel, yin_wheel)


def _enc_kernel(x_ref, cyc_ref, pe_ref, o_ref):
    i = pl.program_id(0)
    base = i * S_TILE
    pos = base + jax.lax.broadcasted_iota(jnp.int32, (S_TILE, 12), 0)
    col = jax.lax.broadcasted_iota(jnp.int32, (S_TILE, 12), 1)
    onehot = (pos % 12 == col).astype(jnp.float32)
    sig = jnp.dot(onehot, cyc_ref[...], preferred_element_type=jnp.float32)
    o_ref[...] = x_ref[...] + (sig + pe_ref[...])[None]


def kernel(x, yang_wheel, yin_wheel, grand_cycle_pe):
    b, s, d = x.shape
    assert s % S_TILE == 0 and d == DIM
    n_tiles = s // S_TILE

    cyc = _sc_gather_cyc(yang_wheel, yin_wheel)

    return pl.pallas_call(
        _enc_kernel,
        grid=(n_tiles, b),
        in_specs=[
            pl.BlockSpec((1, S_TILE, d), lambda i, j: (j, i, 0)),
            pl.BlockSpec((12, DIM), lambda i, j: (0, 0)),
            pl.BlockSpec((S_TILE, d), lambda i, j: (i, 0)),
        ],
        out_specs=pl.BlockSpec((1, S_TILE, d), lambda i, j: (j, i, 0)),
        out_shape=jax.ShapeDtypeStruct((b, s, d), x.dtype),
        compiler_params=pltpu.CompilerParams(
            dimension_semantics=("arbitrary", "arbitrary"),
        ),
    )(x, cyc, grand_cycle_pe)


# speedup vs baseline: 1.0284x; 1.0284x over previous
import functools

import jax
import jax.numpy as jnp
from jax import lax
from jax.experimental import pallas as pl
from jax.experimental.pallas import tpu as pltpu
from jax.experimental.pallas import tpu_sc as plsc

DIM = 2048
HALF = 1024
S_TILE = 1024


def _gather_body(yang_hbm, yin_hbm, cyc_hbm, yang_v, yin_v, cyc_v):
    c = lax.axis_index("c")
    s = lax.axis_index("s")
    wid = s * 2 + c

    @pl.when(wid == 0)
    def _():
        pltpu.sync_copy(yang_hbm, yang_v)
        pltpu.sync_copy(yin_hbm, yin_v)
        for i in range(12):
            i6 = (i + 6) % 12

            @plsc.parallel_loop(0, HALF // 16, unroll=8)
            def _(k, i=i, i6=i6):
                off = k * 16
                cyc_v[i, pl.ds(off, 16)] = yang_v[i, pl.ds(off, 16)]
                cyc_v[i, pl.ds(HALF + off, 16)] = yin_v[i6, pl.ds(off, 16)]

        pltpu.sync_copy(cyc_v, cyc_hbm)


def _sc_gather_cyc(yang_wheel, yin_wheel):
    mesh = plsc.VectorSubcoreMesh(core_axis_name="c", subcore_axis_name="s",
                                  num_cores=1, num_subcores=1)
    k = functools.partial(
        pl.kernel,
        mesh=mesh,
        out_type=jax.ShapeDtypeStruct((12, DIM), jnp.float32),
        scratch_types=[
            pltpu.VMEM((12, HALF), jnp.float32),
            pltpu.VMEM((12, HALF), jnp.float32),
            pltpu.VMEM((12, DIM), jnp.float32),
        ],
    )(_gather_body)
    return k(yang_wheel, yin_wheel)


def _enc_kernel(x_ref, cyc_ref, pe_ref, o_ref):
    i = pl.program_id(0)
    base = i * S_TILE
    pos = base + jax.lax.broadcasted_iota(jnp.int32, (S_TILE, 12), 0)
    col = jax.lax.broadcasted_iota(jnp.int32, (S_TILE, 12), 1)
    onehot = (pos % 12 == col).astype(jnp.float32)
    sig = jnp.dot(onehot, cyc_ref[...], preferred_element_type=jnp.float32)
    o_ref[...] = x_ref[...] + (sig + pe_ref[...])[None]


def kernel(x, yang_wheel, yin_wheel, grand_cycle_pe):
    b, s, d = x.shape
    assert s % S_TILE == 0 and d == DIM
    n_tiles = s // S_TILE

    cyc = _sc_gather_cyc(yang_wheel, yin_wheel)

    return pl.pallas_call(
        _enc_kernel,
        grid=(n_tiles, b),
        in_specs=[
            pl.BlockSpec((1, S_TILE, d), lambda i, j: (j, i, 0)),
            pl.BlockSpec((12, DIM), lambda i, j: (0, 0)),
            pl.BlockSpec((S_TILE, d), lambda i, j: (i, 0)),
        ],
        out_specs=pl.BlockSpec((1, S_TILE, d), lambda i, j: (j, i, 0)),
        out_shape=jax.ShapeDtypeStruct((b, s, d), x.dtype),
        compiler_params=pltpu.CompilerParams(
            dimension_semantics=("parallel", "arbitrary"),
        ),
    )(x, cyc, grand_cycle_pe)
